# baseline (speedup 1.0000x reference)
import jax
import jax.numpy as jnp
from jax import lax
from jax.experimental import pallas as pl
from jax.experimental.pallas import tpu as pltpu
from jax.experimental.pallas import tpu_sc as plsc

N_NODES = 10000
N_EDGES = 320000
NODE_DIM = 128
EDGE_DIM = 16

NUM_CORES = 2
NUM_SUBCORES = 16
NUM_WORKERS = NUM_CORES * NUM_SUBCORES

GATHER_W = 128
CHUNKS = 8
E_PAD = 327680
E_CHUNK = E_PAD // CHUNKS
EDGE_TILE = 1280
TILES_PER_CHUNK = E_CHUNK // EDGE_TILE
CHUNK_REAL = [E_CHUNK] * (CHUNKS - 1) + [N_EDGES - (CHUNKS - 1) * E_CHUNK]

SCAT_CH = 80
N_AGG = 10240
AGG_ROWS = N_AGG // NUM_SUBCORES
ZCH_ROWS = 40

_vector_mesh = plsc.VectorSubcoreMesh(
    core_axis_name="core", subcore_axis_name="subcore")


def _prep_body(nodes_ref, ws_ref, wd_ref, ps_ref, pd_ref):
    x = nodes_ref[...]
    ps_ref[...] = jnp.dot(x, ws_ref[...], preferred_element_type=jnp.float32)
    pd_ref[...] = jnp.dot(x, wd_ref[...], preferred_element_type=jnp.float32)


def _tc_prep(nodes, ws, wd):
    return pl.pallas_call(
        _prep_body,
        out_shape=[
            jax.ShapeDtypeStruct((N_NODES, NODE_DIM), jnp.float32),
            jax.ShapeDtypeStruct((N_NODES, NODE_DIM), jnp.float32),
        ],
    )(nodes, ws, wd)


def _gather_kernel(ps_hbm, pd_hbm, is_hbm, ir_hbm, g1_hbm, g2_hbm,
                   sem1, sem2):
    def body(is_v, ir_v, g1_v, g2_v):
        d1 = pltpu.async_copy(ps_hbm.at[is_v.at[0]], g1_v, sem1)
        d2 = pltpu.async_copy(pd_hbm.at[ir_v.at[0]], g2_v, sem2)
        d1.wait()
        d2.wait()

    pltpu.emit_pipeline(
        body,
        grid=(E_CHUNK // GATHER_W,),
        in_specs=[
            pl.BlockSpec((1, GATHER_W), lambda i: (0, i)),
            pl.BlockSpec((1, GATHER_W), lambda i: (0, i)),
        ],
        out_specs=[
            pl.BlockSpec((GATHER_W, NODE_DIM), lambda i: (i, 0)),
            pl.BlockSpec((GATHER_W, NODE_DIM), lambda i: (i, 0)),
        ],
        core_axis_name=("core", "subcore"),
        dimension_semantics=(pltpu.PARALLEL,),
    )(is_hbm, ir_hbm, g1_hbm, g2_hbm)


def _sc_gather(ps, pd, senders_c, receivers_c):
    k = pl.kernel(
        _gather_kernel,
        out_type=[
            jax.ShapeDtypeStruct((E_CHUNK, NODE_DIM), jnp.float32),
            jax.ShapeDtypeStruct((E_CHUNK, NODE_DIM), jnp.float32),
        ],
        mesh=_vector_mesh,
        scratch_types=[pltpu.SemaphoreType.DMA, pltpu.SemaphoreType.DMA],
    )
    return k(ps, pd, senders_c, receivers_c)


def _edge_body(e_ref, g1_ref, g2_ref, we_ref, w1_ref, w2_ref,
               b0_ref, b1_ref, b2_ref, ne_ref):
    e = e_ref[...]
    h0 = (jnp.dot(e.astype(jnp.bfloat16), we_ref[...].astype(jnp.bfloat16),
                  preferred_element_type=jnp.float32)
          + g1_ref[...] + g2_ref[...] + b0_ref[...])
    h1 = jnp.maximum(h0, 0.0).astype(jnp.bfloat16)
    h2 = jnp.maximum(
        jnp.dot(h1, w1_ref[...].astype(jnp.bfloat16),
                preferred_element_type=jnp.float32)
        + b1_ref[...], 0.0).astype(jnp.bfloat16)
    de = (jnp.dot(h2, w2_ref[...].astype(jnp.bfloat16),
                  preferred_element_type=jnp.float32) + b2_ref[...])
    ne_ref[...] = e + de


def _tc_edge(e_arr, e_tile_off, ntiles, g1, g2, we, w1, w2, b0, b1, b2):
    return pl.pallas_call(
        _edge_body,
        grid=(ntiles,),
        in_specs=[
            pl.BlockSpec((EDGE_TILE, EDGE_DIM),
                         lambda i, o=e_tile_off: (o + i, 0)),
            pl.BlockSpec((EDGE_TILE, NODE_DIM), lambda i: (i, 0)),
            pl.BlockSpec((EDGE_TILE, NODE_DIM), lambda i: (i, 0)),
            pl.BlockSpec((EDGE_DIM, NODE_DIM), lambda i: (0, 0)),
            pl.BlockSpec((NODE_DIM, NODE_DIM), lambda i: (0, 0)),
            pl.BlockSpec((NODE_DIM, EDGE_DIM), lambda i: (0, 0)),
            pl.BlockSpec((1, NODE_DIM), lambda i: (0, 0)),
            pl.BlockSpec((1, NODE_DIM), lambda i: (0, 0)),
            pl.BlockSpec((1, EDGE_DIM), lambda i: (0, 0)),
        ],
        out_specs=pl.BlockSpec((EDGE_TILE, EDGE_DIM), lambda i: (i, 0)),
        out_shape=jax.ShapeDtypeStruct((ntiles * EDGE_TILE, EDGE_DIM),
                                       jnp.float32),
    )(e_arr, g1, g2, we, w1, w2, b0, b1, b2)


def _scatter_kernel(e0, e1, e2, e3, e4, e5, e6, e7, r_hbm, out_hbm,
                    idx_v, e_v, z_v, agg_sh, sem):
    c = lax.axis_index("core")
    s = lax.axis_index("subcore")
    w = c * NUM_SUBCORES + s

    @pl.loop(0, ZCH_ROWS)
    def _(i):
        z_v[i, :] = jnp.zeros((EDGE_DIM,), jnp.float32)

    @pl.loop(0, AGG_ROWS // ZCH_ROWS)
    def _(i):
        pltpu.sync_copy(
            z_v, agg_sh.at[pl.ds(s * AGG_ROWS + i * ZCH_ROWS, ZCH_ROWS)])

    plsc.subcore_barrier()

    for ci, e_hbm in enumerate((e0, e1, e2, e3, e4, e5, e6, e7)):
        per_w = CHUNK_REAL[ci] // NUM_WORKERS
        ebase = w * per_w
        rbase = ci * E_CHUNK + ebase

        @pl.loop(0, per_w // SCAT_CH)
        def _(j):
            pltpu.sync_copy(r_hbm.at[pl.ds(rbase + j * SCAT_CH, SCAT_CH)],
                            idx_v)
            pltpu.sync_copy(e_hbm.at[pl.ds(ebase + j * SCAT_CH, SCAT_CH)],
                            e_v)
            pltpu.sync_copy(e_v, agg_sh.at[idx_v], add=True)

    plsc.subcore_barrier()
    pltpu.sync_copy(agg_sh.at[pl.ds(s * AGG_ROWS, AGG_ROWS)],
                    out_hbm.at[c, pl.ds(s * AGG_ROWS, AGG_ROWS)])


def _sc_scatter(e_chunks, receivers):
    k = pl.kernel(
        _scatter_kernel,
        out_type=jax.ShapeDtypeStruct((NUM_CORES, N_AGG, EDGE_DIM),
                                      jnp.float32),
        mesh=_vector_mesh,
        scratch_types=[
            pltpu.VMEM((SCAT_CH,), jnp.int32),
            pltpu.VMEM((SCAT_CH, EDGE_DIM), jnp.float32),
            pltpu.VMEM((ZCH_ROWS, EDGE_DIM), jnp.float32),
            pltpu.VMEM_SHARED((N_AGG, EDGE_DIM), jnp.float32),
            pltpu.SemaphoreType.DMA,
        ],
        compiler_params=pltpu.CompilerParams(use_tc_tiling_on_sc=False),
    )
    return k(*e_chunks, receivers)


def _node_mlp(nodes_ref, agg2_ref, wx_ref, wa_ref, w1_ref, w2_ref,
              b0_ref, b1_ref, b2_ref):
    nodes = nodes_ref[...]
    agg = (agg2_ref[0] + agg2_ref[1])[:N_NODES, :]
    h = jnp.maximum(
        jnp.dot(nodes, wx_ref[...], preferred_element_type=jnp.float32)
        + jnp.dot(agg, wa_ref[...], preferred_element_type=jnp.float32)
        + b0_ref[...], 0.0)
    h = jnp.maximum(
        jnp.dot(h, w1_ref[...], preferred_element_type=jnp.float32)
        + b1_ref[...], 0.0)
    dn = jnp.dot(h, w2_ref[...], preferred_element_type=jnp.float32) + b2_ref[...]
    return nodes + dn


def _node_body(nodes_ref, agg2_ref, wx_ref, wa_ref, w1_ref, w2_ref,
               b0_ref, b1_ref, b2_ref, nn_ref):
    nn_ref[...] = _node_mlp(nodes_ref, agg2_ref, wx_ref, wa_ref, w1_ref,
                            w2_ref, b0_ref, b1_ref, b2_ref)


def _node_prep_body(nodes_ref, agg2_ref, wx_ref, wa_ref, w1_ref, w2_ref,
                    b0_ref, b1_ref, b2_ref, wsn_ref, wdn_ref,
                    nn_ref, ps_ref, pd_ref):
    nn = _node_mlp(nodes_ref, agg2_ref, wx_ref, wa_ref, w1_ref, w2_ref,
                   b0_ref, b1_ref, b2_ref)
    nn_ref[...] = nn
    ps_ref[...] = jnp.dot(nn, wsn_ref[...], preferred_element_type=jnp.float32)
    pd_ref[...] = jnp.dot(nn, wdn_ref[...], preferred_element_type=jnp.float32)


def _tc_node(nodes, agg2, wx, wa, w1, w2, b0, b1, b2):
    return pl.pallas_call(
        _node_body,
        out_shape=jax.ShapeDtypeStruct((N_NODES, NODE_DIM), jnp.float32),
    )(nodes, agg2, wx, wa, w1, w2, b0, b1, b2)


def _tc_node_prep(nodes, agg2, wx, wa, w1, w2, b0, b1, b2, wsn, wdn):
    return pl.pallas_call(
        _node_prep_body,
        out_shape=[
            jax.ShapeDtypeStruct((N_NODES, NODE_DIM), jnp.float32),
            jax.ShapeDtypeStruct((N_NODES, NODE_DIM), jnp.float32),
            jax.ShapeDtypeStruct((N_NODES, NODE_DIM), jnp.float32),
        ],
    )(nodes, agg2, wx, wa, w1, w2, b0, b1, b2, wsn, wdn)


def kernel(node_emb, edge_index, edge_attr, params):
    B, N, D = node_emb.shape
    nodes = node_emb.reshape(N, D)
    edges = edge_attr.reshape(-1, edge_attr.shape[-1])
    pad = E_PAD - N_EDGES
    spread = (jnp.arange(pad, dtype=jnp.int32) * 13) % N_NODES
    senders_p = jnp.concatenate([edge_index[0], spread]).reshape(1, E_PAD)
    receivers_p = jnp.concatenate([edge_index[1], spread]).reshape(1, E_PAD)
    receivers = edge_index[1]
    s_chunks = [senders_p[:, c * E_CHUNK:(c + 1) * E_CHUNK]
                for c in range(CHUNKS)]
    r_chunks = [receivers_p[:, c * E_CHUNK:(c + 1) * E_CHUNK]
                for c in range(CHUNKS)]

    blocks = params["blocks"]
    sliced = []
    for blk in blocks:
        ew0, ew1, ew2 = blk["ew"]
        eb0, eb1, eb2 = blk["eb"]
        nw0, nw1, nw2 = blk["nw"]
        nb0, nb1, nb2 = blk["nb"]
        sliced.append(dict(
            we=ew0[:EDGE_DIM],
            ws=ew0[EDGE_DIM:EDGE_DIM + NODE_DIM],
            wd=ew0[EDGE_DIM + NODE_DIM:],
            ew1=ew1, ew2=ew2,
            eb0=eb0.reshape(1, -1), eb1=eb1.reshape(1, -1),
            eb2=eb2.reshape(1, -1),
            wx=nw0[:NODE_DIM], wa=nw0[NODE_DIM:],
            nw1=nw1, nw2=nw2,
            nb0=nb0.reshape(1, -1), nb1=nb1.reshape(1, -1),
            nb2=nb2.reshape(1, -1),
        ))

    ps, pd = _tc_prep(nodes, sliced[0]["ws"], sliced[0]["wd"])
    e_chunks = None
    for i, sl in enumerate(sliced):
        new_chunks = []
        for c in range(CHUNKS):
            g1, g2 = _sc_gather(ps, pd, s_chunks[c], r_chunks[c])
            ntiles = CHUNK_REAL[c] // EDGE_TILE
            if e_chunks is None:
                e_arr, off = edges, c * (E_CHUNK // EDGE_TILE)
            else:
                e_arr, off = e_chunks[c], 0
            new_chunks.append(
                _tc_edge(e_arr, off, ntiles, g1, g2,
                         sl["we"], sl["ew1"], sl["ew2"],
                         sl["eb0"], sl["eb1"], sl["eb2"]))
        e_chunks = new_chunks
        agg2 = _sc_scatter(e_chunks, receivers)
        if i + 1 < len(sliced):
            nxt = sliced[i + 1]
            nodes, ps, pd = _tc_node_prep(
                nodes, agg2, sl["wx"], sl["wa"], sl["nw1"], sl["nw2"],
                sl["nb0"], sl["nb1"], sl["nb2"], nxt["ws"], nxt["wd"])
        else:
            nodes = _tc_node(
                nodes, agg2, sl["wx"], sl["wa"], sl["nw1"], sl["nw2"],
                sl["nb0"], sl["nb1"], sl["nb2"])

    return nodes.reshape(B, N, D)


# speedup vs baseline: 1.0840x; 1.0840x over previous
import jax
import jax.numpy as jnp
from jax import lax
from jax.experimental import pallas as pl
from jax.experimental.pallas import tpu as pltpu
from jax.experimental.pallas import tpu_sc as plsc

N_NODES = 10000
N_EDGES = 320000
NODE_DIM = 128
EDGE_DIM = 16

NUM_CORES = 2
NUM_SUBCORES = 16
NUM_WORKERS = NUM_CORES * NUM_SUBCORES

GATHER_W = 128
CHUNKS = 8
E_PAD = 327680
E_CHUNK = E_PAD // CHUNKS
EDGE_TILE = 1280
TILES_PER_CHUNK = E_CHUNK // EDGE_TILE
CHUNK_REAL = [E_CHUNK] * (CHUNKS - 1) + [N_EDGES - (CHUNKS - 1) * E_CHUNK]

SCAT_CH = 80
N_AGG = 10240
AGG_ROWS = N_AGG // NUM_SUBCORES
ZCH_ROWS = 40

_vector_mesh = plsc.VectorSubcoreMesh(
    core_axis_name="core", subcore_axis_name="subcore")


def _prep_body(nodes_ref, ws_ref, wd_ref, ps_ref, pd_ref):
    x = nodes_ref[...]
    ps_ref[...] = jnp.dot(x, ws_ref[...], preferred_element_type=jnp.float32)
    pd_ref[...] = jnp.dot(x, wd_ref[...], preferred_element_type=jnp.float32)


def _tc_prep(nodes, ws, wd):
    return pl.pallas_call(
        _prep_body,
        out_shape=[
            jax.ShapeDtypeStruct((N_NODES, NODE_DIM), jnp.float32),
            jax.ShapeDtypeStruct((N_NODES, NODE_DIM), jnp.float32),
        ],
    )(nodes, ws, wd)


def _gather_kernel(ps_hbm, pd_hbm, is_hbm, ir_hbm, g1_hbm, g2_hbm,
                   sem1, sem2):
    def body(is_v, ir_v, g1_v, g2_v):
        d1 = pltpu.async_copy(ps_hbm.at[is_v.at[0]], g1_v, sem1)
        d2 = pltpu.async_copy(pd_hbm.at[ir_v.at[0]], g2_v, sem2)
        d1.wait()
        d2.wait()

    pltpu.emit_pipeline(
        body,
        grid=(E_CHUNK // GATHER_W,),
        in_specs=[
            pl.BlockSpec((1, GATHER_W), lambda i: (0, i)),
            pl.BlockSpec((1, GATHER_W), lambda i: (0, i)),
        ],
        out_specs=[
            pl.BlockSpec((GATHER_W, NODE_DIM), lambda i: (i, 0)),
            pl.BlockSpec((GATHER_W, NODE_DIM), lambda i: (i, 0)),
        ],
        core_axis_name=("core", "subcore"),
        dimension_semantics=(pltpu.PARALLEL,),
    )(is_hbm, ir_hbm, g1_hbm, g2_hbm)


def _sc_gather(ps, pd, senders_c, receivers_c):
    k = pl.kernel(
        _gather_kernel,
        out_type=[
            jax.ShapeDtypeStruct((E_CHUNK, NODE_DIM), jnp.float32),
            jax.ShapeDtypeStruct((E_CHUNK, NODE_DIM), jnp.float32),
        ],
        mesh=_vector_mesh,
        scratch_types=[pltpu.SemaphoreType.DMA, pltpu.SemaphoreType.DMA],
    )
    return k(ps, pd, senders_c, receivers_c)


def _edge_body(e_ref, g1_ref, g2_ref, we_ref, w1_ref, w2_ref,
               b0_ref, b1_ref, b2_ref, ne_ref):
    e = e_ref[...]
    h0 = (jnp.dot(e.astype(jnp.bfloat16), we_ref[...].astype(jnp.bfloat16),
                  preferred_element_type=jnp.float32)
          + g1_ref[...] + g2_ref[...] + b0_ref[...])
    h1 = jnp.maximum(h0, 0.0).astype(jnp.bfloat16)
    h2 = jnp.maximum(
        jnp.dot(h1, w1_ref[...].astype(jnp.bfloat16),
                preferred_element_type=jnp.float32)
        + b1_ref[...], 0.0).astype(jnp.bfloat16)
    de = (jnp.dot(h2, w2_ref[...].astype(jnp.bfloat16),
                  preferred_element_type=jnp.float32) + b2_ref[...])
    ne_ref[...] = e + de


def _tc_edge(e_arr, e_tile_off, ntiles, g1, g2, we, w1, w2, b0, b1, b2):
    return pl.pallas_call(
        _edge_body,
        grid=(ntiles,),
        in_specs=[
            pl.BlockSpec((EDGE_TILE, EDGE_DIM),
                         lambda i, o=e_tile_off: (o + i, 0)),
            pl.BlockSpec((EDGE_TILE, NODE_DIM), lambda i: (i, 0)),
            pl.BlockSpec((EDGE_TILE, NODE_DIM), lambda i: (i, 0)),
            pl.BlockSpec((EDGE_DIM, NODE_DIM), lambda i: (0, 0)),
            pl.BlockSpec((NODE_DIM, NODE_DIM), lambda i: (0, 0)),
            pl.BlockSpec((NODE_DIM, EDGE_DIM), lambda i: (0, 0)),
            pl.BlockSpec((1, NODE_DIM), lambda i: (0, 0)),
            pl.BlockSpec((1, NODE_DIM), lambda i: (0, 0)),
            pl.BlockSpec((1, EDGE_DIM), lambda i: (0, 0)),
        ],
        out_specs=pl.BlockSpec((EDGE_TILE, EDGE_DIM), lambda i: (i, 0)),
        out_shape=jax.ShapeDtypeStruct((ntiles * EDGE_TILE, EDGE_DIM),
                                       jnp.float32),
    )(e_arr, g1, g2, we, w1, w2, b0, b1, b2)


def _make_scatter_kernel(ci_base):
    def _scatter_kernel(e0, e1, e2, e3, r_hbm, out_hbm,
                        idx_v, e_v, z_v, agg_sh, sem):
        c = lax.axis_index("core")
        s = lax.axis_index("subcore")
        w = c * NUM_SUBCORES + s

        @pl.loop(0, ZCH_ROWS)
        def _(i):
            z_v[i, :] = jnp.zeros((EDGE_DIM,), jnp.float32)

        @pl.loop(0, AGG_ROWS // ZCH_ROWS)
        def _(i):
            pltpu.sync_copy(
                z_v, agg_sh.at[pl.ds(s * AGG_ROWS + i * ZCH_ROWS, ZCH_ROWS)])

        plsc.subcore_barrier()

        for k, e_hbm in enumerate((e0, e1, e2, e3)):
            ci = ci_base + k
            per_w = CHUNK_REAL[ci] // NUM_WORKERS
            ebase = w * per_w
            rbase = ci * E_CHUNK + ebase

            @pl.loop(0, per_w // SCAT_CH)
            def _(j):
                pltpu.sync_copy(r_hbm.at[pl.ds(rbase + j * SCAT_CH, SCAT_CH)],
                                idx_v)
                pltpu.sync_copy(e_hbm.at[pl.ds(ebase + j * SCAT_CH, SCAT_CH)],
                                e_v)
                pltpu.sync_copy(e_v, agg_sh.at[idx_v], add=True)

        plsc.subcore_barrier()
        pltpu.sync_copy(agg_sh.at[pl.ds(s * AGG_ROWS, AGG_ROWS)],
                        out_hbm.at[c, pl.ds(s * AGG_ROWS, AGG_ROWS)])

    return _scatter_kernel


def _sc_scatter(e_chunks, receivers, ci_base):
    k = pl.kernel(
        _make_scatter_kernel(ci_base),
        out_type=jax.ShapeDtypeStruct((NUM_CORES, N_AGG, EDGE_DIM),
                                      jnp.float32),
        mesh=_vector_mesh,
        scratch_types=[
            pltpu.VMEM((SCAT_CH,), jnp.int32),
            pltpu.VMEM((SCAT_CH, EDGE_DIM), jnp.float32),
            pltpu.VMEM((ZCH_ROWS, EDGE_DIM), jnp.float32),
            pltpu.VMEM_SHARED((N_AGG, EDGE_DIM), jnp.float32),
            pltpu.SemaphoreType.DMA,
        ],
        compiler_params=pltpu.CompilerParams(use_tc_tiling_on_sc=False),
    )
    return k(*e_chunks, receivers)


def _node_mlp(nodes_ref, agg2a_ref, agg2b_ref, wx_ref, wa_ref, w1_ref,
              w2_ref, b0_ref, b1_ref, b2_ref):
    nodes = nodes_ref[...]
    agg = (agg2a_ref[0] + agg2a_ref[1]
           + agg2b_ref[0] + agg2b_ref[1])[:N_NODES, :]
    h = jnp.maximum(
        jnp.dot(nodes, wx_ref[...], preferred_element_type=jnp.float32)
        + jnp.dot(agg, wa_ref[...], preferred_element_type=jnp.float32)
        + b0_ref[...], 0.0)
    h = jnp.maximum(
        jnp.dot(h, w1_ref[...], preferred_element_type=jnp.float32)
        + b1_ref[...], 0.0)
    dn = jnp.dot(h, w2_ref[...], preferred_element_type=jnp.float32) + b2_ref[...]
    return nodes + dn


def _node_body(nodes_ref, agg2a_ref, agg2b_ref, wx_ref, wa_ref, w1_ref,
               w2_ref, b0_ref, b1_ref, b2_ref, nn_ref):
    nn_ref[...] = _node_mlp(nodes_ref, agg2a_ref, agg2b_ref, wx_ref, wa_ref,
                            w1_ref, w2_ref, b0_ref, b1_ref, b2_ref)


def _node_prep_body(nodes_ref, agg2a_ref, agg2b_ref, wx_ref, wa_ref, w1_ref,
                    w2_ref, b0_ref, b1_ref, b2_ref, wsn_ref, wdn_ref,
                    nn_ref, ps_ref, pd_ref):
    nn = _node_mlp(nodes_ref, agg2a_ref, agg2b_ref, wx_ref, wa_ref, w1_ref,
                   w2_ref, b0_ref, b1_ref, b2_ref)
    nn_ref[...] = nn
    ps_ref[...] = jnp.dot(nn, wsn_ref[...], preferred_element_type=jnp.float32)
    pd_ref[...] = jnp.dot(nn, wdn_ref[...], preferred_element_type=jnp.float32)


def _tc_node(nodes, agg2a, agg2b, wx, wa, w1, w2, b0, b1, b2):
    return pl.pallas_call(
        _node_body,
        out_shape=jax.ShapeDtypeStruct((N_NODES, NODE_DIM), jnp.float32),
    )(nodes, agg2a, agg2b, wx, wa, w1, w2, b0, b1, b2)


def _tc_node_prep(nodes, agg2a, agg2b, wx, wa, w1, w2, b0, b1, b2, wsn, wdn):
    return pl.pallas_call(
        _node_prep_body,
        out_shape=[
            jax.ShapeDtypeStruct((N_NODES, NODE_DIM), jnp.float32),
            jax.ShapeDtypeStruct((N_NODES, NODE_DIM), jnp.float32),
            jax.ShapeDtypeStruct((N_NODES, NODE_DIM), jnp.float32),
        ],
    )(nodes, agg2a, agg2b, wx, wa, w1, w2, b0, b1, b2, wsn, wdn)


def kernel(node_emb, edge_index, edge_attr, params):
    B, N, D = node_emb.shape
    nodes = node_emb.reshape(N, D)
    edges = edge_attr.reshape(-1, edge_attr.shape[-1])
    pad = E_PAD - N_EDGES
    spread = (jnp.arange(pad, dtype=jnp.int32) * 13) % N_NODES
    senders_p = jnp.concatenate([edge_index[0], spread]).reshape(1, E_PAD)
    receivers_p = jnp.concatenate([edge_index[1], spread]).reshape(1, E_PAD)
    receivers = edge_index[1]
    s_chunks = [senders_p[:, c * E_CHUNK:(c + 1) * E_CHUNK]
                for c in range(CHUNKS)]
    r_chunks = [receivers_p[:, c * E_CHUNK:(c + 1) * E_CHUNK]
                for c in range(CHUNKS)]

    blocks = params["blocks"]
    sliced = []
    for blk in blocks:
        ew0, ew1, ew2 = blk["ew"]
        eb0, eb1, eb2 = blk["eb"]
        nw0, nw1, nw2 = blk["nw"]
        nb0, nb1, nb2 = blk["nb"]
        sliced.append(dict(
            we=ew0[:EDGE_DIM],
            ws=ew0[EDGE_DIM:EDGE_DIM + NODE_DIM],
            wd=ew0[EDGE_DIM + NODE_DIM:],
            ew1=ew1, ew2=ew2,
            eb0=eb0.reshape(1, -1), eb1=eb1.reshape(1, -1),
            eb2=eb2.reshape(1, -1),
            wx=nw0[:NODE_DIM], wa=nw0[NODE_DIM:],
            nw1=nw1, nw2=nw2,
            nb0=nb0.reshape(1, -1), nb1=nb1.reshape(1, -1),
            nb2=nb2.reshape(1, -1),
        ))

    ps, pd = _tc_prep(nodes, sliced[0]["ws"], sliced[0]["wd"])
    e_chunks = None
    for i, sl in enumerate(sliced):
        new_chunks = []
        for c in range(CHUNKS):
            g1, g2 = _sc_gather(ps, pd, s_chunks[c], r_chunks[c])
            ntiles = CHUNK_REAL[c] // EDGE_TILE
            if e_chunks is None:
                e_arr, off = edges, c * (E_CHUNK // EDGE_TILE)
            else:
                e_arr, off = e_chunks[c], 0
            new_chunks.append(
                _tc_edge(e_arr, off, ntiles, g1, g2,
                         sl["we"], sl["ew1"], sl["ew2"],
                         sl["eb0"], sl["eb1"], sl["eb2"]))
        e_chunks = new_chunks
        agg2a = _sc_scatter(e_chunks[:CHUNKS // 2], receivers, 0)
        agg2b = _sc_scatter(e_chunks[CHUNKS // 2:], receivers, CHUNKS // 2)
        if i + 1 < len(sliced):
            nxt = sliced[i + 1]
            nodes, ps, pd = _tc_node_prep(
                nodes, agg2a, agg2b, sl["wx"], sl["wa"], sl["nw1"],
                sl["nw2"], sl["nb0"], sl["nb1"], sl["nb2"],
                nxt["ws"], nxt["wd"])
        else:
            nodes = _tc_node(
                nodes, agg2a, agg2b, sl["wx"], sl["wa"], sl["nw1"],
                sl["nw2"], sl["nb0"], sl["nb1"], sl["nb2"])

    return nodes.reshape(B, N, D)


# speedup vs baseline: 1.1748x; 1.0837x over previous
import jax
import jax.numpy as jnp
from jax import lax
from jax.experimental import pallas as pl
from jax.experimental.pallas import tpu as pltpu
from jax.experimental.pallas import tpu_sc as plsc

N_NODES = 10000
N_EDGES = 320000
NODE_DIM = 128
EDGE_DIM = 16

NUM_CORES = 2
NUM_SUBCORES = 16
NUM_WORKERS = NUM_CORES * NUM_SUBCORES

GATHER_W = 128
CHUNKS = 8
E_PAD = 327680
E_CHUNK = E_PAD // CHUNKS
EDGE_TILE = 2560
TILES_PER_CHUNK = E_CHUNK // EDGE_TILE
CHUNK_REAL = [E_CHUNK] * (CHUNKS - 1) + [N_EDGES - (CHUNKS - 1) * E_CHUNK]

SCAT_CH = 80
N_AGG = 10240
AGG_ROWS = N_AGG // NUM_SUBCORES
ZCH_ROWS = 40

_vector_mesh = plsc.VectorSubcoreMesh(
    core_axis_name="core", subcore_axis_name="subcore")


def _prep_body(nodes_ref, ws_ref, wd_ref, ps_ref, pd_ref):
    x = nodes_ref[...]
    ps_ref[...] = jnp.dot(x, ws_ref[...], preferred_element_type=jnp.float32)
    pd_ref[...] = jnp.dot(x, wd_ref[...], preferred_element_type=jnp.float32)


def _tc_prep(nodes, ws, wd):
    return pl.pallas_call(
        _prep_body,
        out_shape=[
            jax.ShapeDtypeStruct((N_NODES, NODE_DIM), jnp.float32),
            jax.ShapeDtypeStruct((N_NODES, NODE_DIM), jnp.float32),
        ],
    )(nodes, ws, wd)


def _gather_kernel(ps_hbm, pd_hbm, is_hbm, ir_hbm, g1_hbm, g2_hbm,
                   sem1, sem2):
    def body(is_v, ir_v, g1_v, g2_v):
        d1 = pltpu.async_copy(ps_hbm.at[is_v.at[0]], g1_v, sem1)
        d2 = pltpu.async_copy(pd_hbm.at[ir_v.at[0]], g2_v, sem2)
        d1.wait()
        d2.wait()

    pltpu.emit_pipeline(
        body,
        grid=(E_CHUNK // GATHER_W,),
        in_specs=[
            pl.BlockSpec((1, GATHER_W), lambda i: (0, i)),
            pl.BlockSpec((1, GATHER_W), lambda i: (0, i)),
        ],
        out_specs=[
            pl.BlockSpec((GATHER_W, NODE_DIM), lambda i: (i, 0)),
            pl.BlockSpec((GATHER_W, NODE_DIM), lambda i: (i, 0)),
        ],
        core_axis_name=("core", "subcore"),
        dimension_semantics=(pltpu.PARALLEL,),
    )(is_hbm, ir_hbm, g1_hbm, g2_hbm)


def _sc_gather(ps, pd, senders_c, receivers_c):
    k = pl.kernel(
        _gather_kernel,
        out_type=[
            jax.ShapeDtypeStruct((E_CHUNK, NODE_DIM), jnp.float32),
            jax.ShapeDtypeStruct((E_CHUNK, NODE_DIM), jnp.float32),
        ],
        mesh=_vector_mesh,
        scratch_types=[pltpu.SemaphoreType.DMA, pltpu.SemaphoreType.DMA],
    )
    return k(ps, pd, senders_c, receivers_c)


def _edge_body(e_ref, g1_ref, g2_ref, we_ref, w1_ref, w2_ref,
               b0_ref, b1_ref, b2_ref, ne_ref):
    e = e_ref[...]
    h0 = (jnp.dot(e.astype(jnp.bfloat16), we_ref[...].astype(jnp.bfloat16),
                  preferred_element_type=jnp.float32)
          + g1_ref[...] + g2_ref[...] + b0_ref[...])
    h1 = jnp.maximum(h0, 0.0).astype(jnp.bfloat16)
    h2 = jnp.maximum(
        jnp.dot(h1, w1_ref[...].astype(jnp.bfloat16),
                preferred_element_type=jnp.float32)
        + b1_ref[...], 0.0).astype(jnp.bfloat16)
    de = (jnp.dot(h2, w2_ref[...].astype(jnp.bfloat16),
                  preferred_element_type=jnp.float32) + b2_ref[...])
    ne_ref[...] = e + de


def _tc_edge(e_arr, e_tile_off, ntiles, g1, g2, we, w1, w2, b0, b1, b2):
    return pl.pallas_call(
        _edge_body,
        grid=(ntiles,),
        in_specs=[
            pl.BlockSpec((EDGE_TILE, EDGE_DIM),
                         lambda i, o=e_tile_off: (o + i, 0)),
            pl.BlockSpec((EDGE_TILE, NODE_DIM), lambda i: (i, 0)),
            pl.BlockSpec((EDGE_TILE, NODE_DIM), lambda i: (i, 0)),
            pl.BlockSpec((EDGE_DIM, NODE_DIM), lambda i: (0, 0)),
            pl.BlockSpec((NODE_DIM, NODE_DIM), lambda i: (0, 0)),
            pl.BlockSpec((NODE_DIM, EDGE_DIM), lambda i: (0, 0)),
            pl.BlockSpec((1, NODE_DIM), lambda i: (0, 0)),
            pl.BlockSpec((1, NODE_DIM), lambda i: (0, 0)),
            pl.BlockSpec((1, EDGE_DIM), lambda i: (0, 0)),
        ],
        out_specs=pl.BlockSpec((EDGE_TILE, EDGE_DIM), lambda i: (i, 0)),
        out_shape=jax.ShapeDtypeStruct((ntiles * EDGE_TILE, EDGE_DIM),
                                       jnp.float32),
    )(e_arr, g1, g2, we, w1, w2, b0, b1, b2)


def _make_scatter_kernel(ci_base):
    def _scatter_kernel(e0, e1, e2, e3, r_hbm, out_hbm,
                        idx_v, e_v, z_v, agg_sh, sem):
        c = lax.axis_index("core")
        s = lax.axis_index("subcore")
        w = c * NUM_SUBCORES + s

        @pl.loop(0, ZCH_ROWS)
        def _(i):
            z_v[i, :] = jnp.zeros((EDGE_DIM,), jnp.float32)

        @pl.loop(0, AGG_ROWS // ZCH_ROWS)
        def _(i):
            pltpu.sync_copy(
                z_v, agg_sh.at[pl.ds(s * AGG_ROWS + i * ZCH_ROWS, ZCH_ROWS)])

        plsc.subcore_barrier()

        for k, e_hbm in enumerate((e0, e1, e2, e3)):
            ci = ci_base + k
            per_w = CHUNK_REAL[ci] // NUM_WORKERS
            ebase = w * per_w
            rbase = ci * E_CHUNK + ebase

            @pl.loop(0, per_w // SCAT_CH)
            def _(j):
                pltpu.sync_copy(r_hbm.at[pl.ds(rbase + j * SCAT_CH, SCAT_CH)],
                                idx_v)
                pltpu.sync_copy(e_hbm.at[pl.ds(ebase + j * SCAT_CH, SCAT_CH)],
                                e_v)
                pltpu.sync_copy(e_v, agg_sh.at[idx_v], add=True)

        plsc.subcore_barrier()
        pltpu.sync_copy(agg_sh.at[pl.ds(s * AGG_ROWS, AGG_ROWS)],
                        out_hbm.at[c, pl.ds(s * AGG_ROWS, AGG_ROWS)])

    return _scatter_kernel


def _sc_scatter(e_chunks, receivers, ci_base):
    k = pl.kernel(
        _make_scatter_kernel(ci_base),
        out_type=jax.ShapeDtypeStruct((NUM_CORES, N_AGG, EDGE_DIM),
                                      jnp.float32),
        mesh=_vector_mesh,
        scratch_types=[
            pltpu.VMEM((SCAT_CH,), jnp.int32),
            pltpu.VMEM((SCAT_CH, EDGE_DIM), jnp.float32),
            pltpu.VMEM((ZCH_ROWS, EDGE_DIM), jnp.float32),
            pltpu.VMEM_SHARED((N_AGG, EDGE_DIM), jnp.float32),
            pltpu.SemaphoreType.DMA,
        ],
        compiler_params=pltpu.CompilerParams(use_tc_tiling_on_sc=False),
    )
    return k(*e_chunks, receivers)


def _node_mlp(nodes_ref, agg2a_ref, agg2b_ref, wx_ref, wa_ref, w1_ref,
              w2_ref, b0_ref, b1_ref, b2_ref):
    nodes = nodes_ref[...]
    agg = (agg2a_ref[0] + agg2a_ref[1]
           + agg2b_ref[0] + agg2b_ref[1])[:N_NODES, :]
    h = jnp.maximum(
        jnp.dot(nodes, wx_ref[...], preferred_element_type=jnp.float32)
        + jnp.dot(agg, wa_ref[...], preferred_element_type=jnp.float32)
        + b0_ref[...], 0.0)
    h = jnp.maximum(
        jnp.dot(h, w1_ref[...], preferred_element_type=jnp.float32)
        + b1_ref[...], 0.0)
    dn = jnp.dot(h, w2_ref[...], preferred_element_type=jnp.float32) + b2_ref[...]
    return nodes + dn


def _node_body(nodes_ref, agg2a_ref, agg2b_ref, wx_ref, wa_ref, w1_ref,
               w2_ref, b0_ref, b1_ref, b2_ref, nn_ref):
    nn_ref[...] = _node_mlp(nodes_ref, agg2a_ref, agg2b_ref, wx_ref, wa_ref,
                            w1_ref, w2_ref, b0_ref, b1_ref, b2_ref)


def _node_prep_body(nodes_ref, agg2a_ref, agg2b_ref, wx_ref, wa_ref, w1_ref,
                    w2_ref, b0_ref, b1_ref, b2_ref, wsn_ref, wdn_ref,
                    nn_ref, ps_ref, pd_ref):
    nn = _node_mlp(nodes_ref, agg2a_ref, agg2b_ref, wx_ref, wa_ref, w1_ref,
                   w2_ref, b0_ref, b1_ref, b2_ref)
    nn_ref[...] = nn
    ps_ref[...] = jnp.dot(nn, wsn_ref[...], preferred_element_type=jnp.float32)
    pd_ref[...] = jnp.dot(nn, wdn_ref[...], preferred_element_type=jnp.float32)


def _tc_node(nodes, agg2a, agg2b, wx, wa, w1, w2, b0, b1, b2):
    return pl.pallas_call(
        _node_body,
        out_shape=jax.ShapeDtypeStruct((N_NODES, NODE_DIM), jnp.float32),
    )(nodes, agg2a, agg2b, wx, wa, w1, w2, b0, b1, b2)


def _tc_node_prep(nodes, agg2a, agg2b, wx, wa, w1, w2, b0, b1, b2, wsn, wdn):
    return pl.pallas_call(
        _node_prep_body,
        out_shape=[
            jax.ShapeDtypeStruct((N_NODES, NODE_DIM), jnp.float32),
            jax.ShapeDtypeStruct((N_NODES, NODE_DIM), jnp.float32),
            jax.ShapeDtypeStruct((N_NODES, NODE_DIM), jnp.float32),
        ],
    )(nodes, agg2a, agg2b, wx, wa, w1, w2, b0, b1, b2, wsn, wdn)


def kernel(node_emb, edge_index, edge_attr, params):
    B, N, D = node_emb.shape
    nodes = node_emb.reshape(N, D)
    edges = edge_attr.reshape(-1, edge_attr.shape[-1])
    pad = E_PAD - N_EDGES
    spread = (jnp.arange(pad, dtype=jnp.int32) * 13) % N_NODES
    senders_p = jnp.concatenate([edge_index[0], spread]).reshape(1, E_PAD)
    receivers_p = jnp.concatenate([edge_index[1], spread]).reshape(1, E_PAD)
    receivers = edge_index[1]
    s_chunks = [senders_p[:, c * E_CHUNK:(c + 1) * E_CHUNK]
                for c in range(CHUNKS)]
    r_chunks = [receivers_p[:, c * E_CHUNK:(c + 1) * E_CHUNK]
                for c in range(CHUNKS)]

    blocks = params["blocks"]
    sliced = []
    for blk in blocks:
        ew0, ew1, ew2 = blk["ew"]
        eb0, eb1, eb2 = blk["eb"]
        nw0, nw1, nw2 = blk["nw"]
        nb0, nb1, nb2 = blk["nb"]
        sliced.append(dict(
            we=ew0[:EDGE_DIM],
            ws=ew0[EDGE_DIM:EDGE_DIM + NODE_DIM],
            wd=ew0[EDGE_DIM + NODE_DIM:],
            ew1=ew1, ew2=ew2,
            eb0=eb0.reshape(1, -1), eb1=eb1.reshape(1, -1),
            eb2=eb2.reshape(1, -1),
            wx=nw0[:NODE_DIM], wa=nw0[NODE_DIM:],
            nw1=nw1, nw2=nw2,
            nb0=nb0.reshape(1, -1), nb1=nb1.reshape(1, -1),
            nb2=nb2.reshape(1, -1),
        ))

    ps, pd = _tc_prep(nodes, sliced[0]["ws"], sliced[0]["wd"])
    e_chunks = None
    for i, sl in enumerate(sliced):
        new_chunks = []
        for c in range(CHUNKS):
            g1, g2 = _sc_gather(ps, pd, s_chunks[c], r_chunks[c])
            ntiles = CHUNK_REAL[c] // EDGE_TILE
            if e_chunks is None:
                e_arr, off = edges, c * (E_CHUNK // EDGE_TILE)
            else:
                e_arr, off = e_chunks[c], 0
            new_chunks.append(
                _tc_edge(e_arr, off, ntiles, g1, g2,
                         sl["we"], sl["ew1"], sl["ew2"],
                         sl["eb0"], sl["eb1"], sl["eb2"]))
        e_chunks = new_chunks
        agg2a = _sc_scatter(e_chunks[:CHUNKS // 2], receivers, 0)
        agg2b = _sc_scatter(e_chunks[CHUNKS // 2:], receivers, CHUNKS // 2)
        if i + 1 < len(sliced):
            nxt = sliced[i + 1]
            nodes, ps, pd = _tc_node_prep(
                nodes, agg2a, agg2b, sl["wx"], sl["wa"], sl["nw1"],
                sl["nw2"], sl["nb0"], sl["nb1"], sl["nb2"],
                nxt["ws"], nxt["wd"])
        else:
            nodes = _tc_node(
                nodes, agg2a, agg2b, sl["wx"], sl["wa"], sl["nw1"],
                sl["nw2"], sl["nb0"], sl["nb1"], sl["nb2"])

    return nodes.reshape(B, N, D)
